# baseline (speedup 1.0000x reference)
import functools

import jax
import jax.numpy as jnp
from jax import lax
from jax.experimental import pallas as pl
from jax.experimental.pallas import tpu as pltpu
from jax.experimental.pallas import tpu_sc as plsc

N = 10000
D = 256
E = 160000
P = 10112
NC = 2
NS = 16
NW = NC * NS
CH = 128
NCHT = E // CH
NCH = 40
NTL = NCHT - (NW - 1) * NCH
VPC = CH // 16
ZCH = 3792
WIN = 8


def _proj_body(x0_ref, x1_ref, ws_ref, wd_ref, atts_ref, attd_ref, out_ref):
    i = pl.program_id(0)
    xs = x0_ref[...]
    xd = x1_ref[...]
    wsa = jnp.dot(ws_ref[...], atts_ref[...], preferred_element_type=jnp.float32)
    wda = jnp.dot(wd_ref[...], attd_ref[...], preferred_element_type=jnp.float32)
    a_s = jnp.dot(xs, wsa, preferred_element_type=jnp.float32)
    a_d = jnp.dot(xd, wda, preferred_element_type=jnp.float32)
    h0 = jnp.dot(xs, ws_ref[:, 0], preferred_element_type=jnp.float32)
    h1 = jnp.dot(xs, ws_ref[:, 1], preferred_element_type=jnp.float32)
    tab = jnp.concatenate(
        [a_s[None, :], a_d[None, :], h0[None, :], h1[None, :]], axis=0)
    out_ref[:, pl.ds(i * 128, 128)] = tab


def _proj(x0, x1, ws, wd, atts, attd):
    return pl.pallas_call(
        _proj_body,
        grid=(P // 128,),
        in_specs=[
            pl.BlockSpec((128, D), lambda i: (i, 0)),
            pl.BlockSpec((128, D), lambda i: (i, 0)),
            pl.BlockSpec((D, 2), lambda i: (0, 0)),
            pl.BlockSpec((D, 2), lambda i: (0, 0)),
            pl.BlockSpec((2,), lambda i: (0,)),
            pl.BlockSpec((2,), lambda i: (0,)),
        ],
        out_specs=pl.BlockSpec((4, P), lambda i: (0, 0)),
        out_shape=jax.ShapeDtypeStruct((4, P), jnp.float32),
    )(x0, x1, ws, wd, atts, attd)


def _edge_body(tab_hbm, ei_hbm, out_hbm,
               tab_v, srcf_v, dstf_v, dst0_v, dstp_v, dst2p_v,
               ex_v, e0_v, e1_v, z_v, acc_sh, sem):
    c = lax.axis_index("c")
    s = lax.axis_index("s")
    wid = s * NC + c
    base = wid * NCH * CH
    last = wid == NW - 1
    ncnt = jnp.where(last, NTL, NCH)

    pltpu.sync_copy(tab_hbm, tab_v)

    @pl.when(jnp.logical_not(last))
    def _full():
        pltpu.sync_copy(ei_hbm.at[0, pl.ds(base, NCH * CH)], srcf_v)
        pltpu.sync_copy(ei_hbm.at[1, pl.ds(base, NCH * CH)], dstf_v)

    @pl.when(last)
    def _tail():
        pltpu.sync_copy(ei_hbm.at[0, pl.ds(base, NTL * CH)],
                        srcf_v.at[pl.ds(0, NTL * CH)])
        pltpu.sync_copy(ei_hbm.at[1, pl.ds(base, NTL * CH)],
                        dstf_v.at[pl.ds(0, NTL * CH)])

    @pl.when(s == 0)
    def _zero():
        def zb(i, carry):
            z_v[pl.ds(i * 16, 16)] = jnp.zeros((16,), jnp.float32)
            return carry
        lax.fori_loop(0, ZCH // 16, zb, 0)

        def zs(i, carry):
            pltpu.sync_copy(z_v, acc_sh.at[pl.ds(i * ZCH, ZCH)])
            return carry
        lax.fori_loop(0, (3 * P) // ZCH, zs, 0)

    plsc.subcore_barrier()

    r0 = jnp.zeros((16,), jnp.int32)
    r1 = r0 + 1
    r2 = r0 + 2
    r3 = r0 + 3

    def chunk(j, carry):
        for k in range(VPC):
            sl = pl.ds(k * 16, 16)
            fsl = pl.ds(j * CH + k * 16, 16)
            src = srcf_v[fsl]
            dst = dstf_v[fsl]
            a_s = plsc.load_gather(tab_v, [r0, src])
            a_d = plsc.load_gather(tab_v, [r1, dst])
            h0 = plsc.load_gather(tab_v, [r2, src])
            h1 = plsc.load_gather(tab_v, [r3, src])
            e = a_s + a_d
            e = jnp.where(e > 0.0, e, 0.2 * e)
            ex = jnp.exp(e)
            ex_v[j, sl] = ex
            e0_v[j, sl] = ex * h0
            e1_v[j, sl] = ex * h1
            dst0_v[j, sl] = dst
            dstp_v[j, sl] = dst + P
            dst2p_v[j, sl] = dst + 2 * P
        pltpu.async_copy(ex_v.at[j], acc_sh.at[dst0_v.at[j]], sem, add=True)
        pltpu.async_copy(e0_v.at[j], acc_sh.at[dstp_v.at[j]], sem, add=True)
        pltpu.async_copy(e1_v.at[j], acc_sh.at[dst2p_v.at[j]], sem, add=True)

        @pl.when(j >= WIN)
        def _drain_one():
            i = j - WIN
            pltpu.make_async_copy(ex_v.at[i], acc_sh.at[dst0_v.at[i]], sem).wait()
            pltpu.make_async_copy(e0_v.at[i], acc_sh.at[dstp_v.at[i]], sem).wait()
            pltpu.make_async_copy(e1_v.at[i], acc_sh.at[dst2p_v.at[i]], sem).wait()
        return carry

    lax.fori_loop(0, ncnt, chunk, 0)

    def draintail(i, carry):
        pltpu.make_async_copy(ex_v.at[i], acc_sh.at[dst0_v.at[i]], sem).wait()
        pltpu.make_async_copy(e0_v.at[i], acc_sh.at[dstp_v.at[i]], sem).wait()
        pltpu.make_async_copy(e1_v.at[i], acc_sh.at[dst2p_v.at[i]], sem).wait()
        return carry

    lax.fori_loop(ncnt - WIN, ncnt, draintail, 0)

    plsc.subcore_barrier()

    @pl.when(s == 0)
    def _flush():
        pltpu.sync_copy(acc_sh, out_hbm.at[c])


_edge_call = functools.partial(
    pl.kernel,
    out_type=jax.ShapeDtypeStruct((NC, 3 * P), jnp.float32),
    mesh=plsc.VectorSubcoreMesh(core_axis_name="c", subcore_axis_name="s"),
    compiler_params=pltpu.CompilerParams(needs_layout_passes=False),
    scratch_types=[
        pltpu.VMEM((4, P), jnp.float32),
        pltpu.VMEM((NCH * CH,), jnp.int32),
        pltpu.VMEM((NCH * CH,), jnp.int32),
        pltpu.VMEM((NCH, CH), jnp.int32),
        pltpu.VMEM((NCH, CH), jnp.int32),
        pltpu.VMEM((NCH, CH), jnp.int32),
        pltpu.VMEM((NCH, CH), jnp.float32),
        pltpu.VMEM((NCH, CH), jnp.float32),
        pltpu.VMEM((NCH, CH), jnp.float32),
        pltpu.VMEM((ZCH,), jnp.float32),
        pltpu.VMEM_SHARED((3 * P,), jnp.float32),
        pltpu.SemaphoreType.DMA,
    ],
)(_edge_body)


def _head_body(parts_ref, ba_ref, w1_ref, b1_ref, w2_ref, b2_ref,
               w3_ref, b3_ref, out_ref):
    p = parts_ref[...]
    tot = p[0] + p[1]
    inv = 1.0 / (tot[0:P] + 1e-16)
    o0 = jnp.maximum(tot[P:2 * P] * inv + ba_ref[0], 0.0)
    o1 = jnp.maximum(tot[2 * P:3 * P] * inv + ba_ref[1], 0.0)
    f1 = [jnp.maximum(o0 * w1_ref[0, j] + o1 * w1_ref[1, j] + b1_ref[j], 0.0)
          for j in range(2)]
    f2 = [jnp.maximum(f1[0] * w2_ref[0, j] + f1[1] * w2_ref[1, j] + b2_ref[j],
                      0.0)
          for j in range(4)]
    z = (f2[0] * w3_ref[0, 0] + f2[1] * w3_ref[1, 0]
         + f2[2] * w3_ref[2, 0] + f2[3] * w3_ref[3, 0] + b3_ref[0])
    out_ref[...] = jax.nn.sigmoid(z)[None, :]


def _head(parts, ba, w1, b1, w2, b2, w3, b3):
    smem = pl.BlockSpec(memory_space=pltpu.SMEM)
    return pl.pallas_call(
        _head_body,
        in_specs=[pl.BlockSpec(memory_space=pltpu.VMEM),
                  smem, smem, smem, smem, smem, smem, smem],
        out_specs=pl.BlockSpec(memory_space=pltpu.VMEM),
        out_shape=jax.ShapeDtypeStruct((1, P), jnp.float32),
    )(parts, ba, w1, b1, w2, b2, w3, b3)


def kernel(x_0, x_1, ei_a, ei_b, W_src_a, W_dst_a, att_src_a, att_dst_a,
           bias_a, W_src_b, W_dst_b, att_src_b, att_dst_b, bias_b,
           W_fc1, b_fc1, W_fc2, b_fc2, W_fc3, b_fc3):
    tab_p = _proj(x_0, x_1, W_src_a, W_dst_a, att_src_a, att_dst_a)
    ei = ei_a.astype(jnp.int32)
    parts = _edge_call(tab_p, ei)
    out = _head(parts, bias_a, W_fc1, b_fc1, W_fc2, b_fc2, W_fc3, b_fc3)
    return out[0, :N].reshape(N, 1)


# speedup vs baseline: 1.9120x; 1.9120x over previous
import functools

import jax
import jax.numpy as jnp
from jax import lax
from jax.experimental import pallas as pl
from jax.experimental.pallas import tpu as pltpu
from jax.experimental.pallas import tpu_sc as plsc

N = 10000
D = 256
E = 160000
P = 10112
NC = 2
NS = 16
NW = NC * NS
CH = 128
NCHT = E // CH
NCH = 40
NTL = NCHT - (NW - 1) * NCH
VPC = CH // 16
ZCH = 3792
WIN = 8


def _proj_body(x0_ref, x1_ref, ws_ref, wd_ref, atts_ref, attd_ref, out_ref):
    xs = x0_ref[...]
    xd = x1_ref[...]
    wsa = jnp.dot(ws_ref[...], atts_ref[...], preferred_element_type=jnp.float32)
    wda = jnp.dot(wd_ref[...], attd_ref[...], preferred_element_type=jnp.float32)
    a_s = jnp.dot(xs, wsa, preferred_element_type=jnp.float32)
    a_d = jnp.dot(xd, wda, preferred_element_type=jnp.float32)
    h0 = jnp.dot(xs, ws_ref[:, 0], preferred_element_type=jnp.float32)
    h1 = jnp.dot(xs, ws_ref[:, 1], preferred_element_type=jnp.float32)
    tab = jnp.concatenate(
        [a_s[None, :], a_d[None, :], h0[None, :], h1[None, :]], axis=0)
    out_ref[...] = jnp.pad(tab, ((0, 0), (0, P - N)))


def _proj(x0, x1, ws, wd, atts, attd):
    return pl.pallas_call(
        _proj_body,
        out_shape=jax.ShapeDtypeStruct((4, P), jnp.float32),
    )(x0, x1, ws, wd, atts, attd)


def _edge_body(tab_hbm, ei_hbm, out_hbm,
               tab_v, srcf_v, dstf_v, dst0_v, dstp_v, dst2p_v,
               ex_v, e0_v, e1_v, z_v, acc_sh, sem):
    c = lax.axis_index("c")
    s = lax.axis_index("s")
    wid = s * NC + c
    base = wid * NCH * CH
    last = wid == NW - 1
    ncnt = jnp.where(last, NTL, NCH)

    pltpu.sync_copy(tab_hbm, tab_v)

    @pl.when(jnp.logical_not(last))
    def _full():
        pltpu.sync_copy(ei_hbm.at[0, pl.ds(base, NCH * CH)], srcf_v)
        pltpu.sync_copy(ei_hbm.at[1, pl.ds(base, NCH * CH)], dstf_v)

    @pl.when(last)
    def _tail():
        pltpu.sync_copy(ei_hbm.at[0, pl.ds(base, NTL * CH)],
                        srcf_v.at[pl.ds(0, NTL * CH)])
        pltpu.sync_copy(ei_hbm.at[1, pl.ds(base, NTL * CH)],
                        dstf_v.at[pl.ds(0, NTL * CH)])

    @pl.when(s == 0)
    def _zero():
        def zb(i, carry):
            z_v[pl.ds(i * 16, 16)] = jnp.zeros((16,), jnp.float32)
            return carry
        lax.fori_loop(0, ZCH // 16, zb, 0)

        def zs(i, carry):
            pltpu.sync_copy(z_v, acc_sh.at[pl.ds(i * ZCH, ZCH)])
            return carry
        lax.fori_loop(0, (3 * P) // ZCH, zs, 0)

    plsc.subcore_barrier()

    r0 = jnp.zeros((16,), jnp.int32)
    r1 = r0 + 1
    r2 = r0 + 2
    r3 = r0 + 3

    def chunk(j, carry):
        for k in range(VPC):
            sl = pl.ds(k * 16, 16)
            fsl = pl.ds(j * CH + k * 16, 16)
            src = srcf_v[fsl]
            dst = dstf_v[fsl]
            a_s = plsc.load_gather(tab_v, [r0, src])
            a_d = plsc.load_gather(tab_v, [r1, dst])
            h0 = plsc.load_gather(tab_v, [r2, src])
            h1 = plsc.load_gather(tab_v, [r3, src])
            e = a_s + a_d
            e = jnp.where(e > 0.0, e, 0.2 * e)
            ex = jnp.exp(e)
            ex_v[j, sl] = ex
            e0_v[j, sl] = ex * h0
            e1_v[j, sl] = ex * h1
            dst0_v[j, sl] = dst
            dstp_v[j, sl] = dst + P
            dst2p_v[j, sl] = dst + 2 * P
        pltpu.async_copy(ex_v.at[j], acc_sh.at[dst0_v.at[j]], sem, add=True)
        pltpu.async_copy(e0_v.at[j], acc_sh.at[dstp_v.at[j]], sem, add=True)
        pltpu.async_copy(e1_v.at[j], acc_sh.at[dst2p_v.at[j]], sem, add=True)

        @pl.when(j >= WIN)
        def _drain_one():
            i = j - WIN
            pltpu.make_async_copy(ex_v.at[i], acc_sh.at[dst0_v.at[i]], sem).wait()
            pltpu.make_async_copy(e0_v.at[i], acc_sh.at[dstp_v.at[i]], sem).wait()
            pltpu.make_async_copy(e1_v.at[i], acc_sh.at[dst2p_v.at[i]], sem).wait()
        return carry

    lax.fori_loop(0, ncnt, chunk, 0)

    def draintail(i, carry):
        pltpu.make_async_copy(ex_v.at[i], acc_sh.at[dst0_v.at[i]], sem).wait()
        pltpu.make_async_copy(e0_v.at[i], acc_sh.at[dstp_v.at[i]], sem).wait()
        pltpu.make_async_copy(e1_v.at[i], acc_sh.at[dst2p_v.at[i]], sem).wait()
        return carry

    lax.fori_loop(ncnt - WIN, ncnt, draintail, 0)

    plsc.subcore_barrier()

    @pl.when(s == 0)
    def _flush():
        pltpu.sync_copy(acc_sh, out_hbm.at[c])


_edge_call = functools.partial(
    pl.kernel,
    out_type=jax.ShapeDtypeStruct((NC, 3 * P), jnp.float32),
    mesh=plsc.VectorSubcoreMesh(core_axis_name="c", subcore_axis_name="s"),
    compiler_params=pltpu.CompilerParams(needs_layout_passes=False),
    scratch_types=[
        pltpu.VMEM((4, P), jnp.float32),
        pltpu.VMEM((NCH * CH,), jnp.int32),
        pltpu.VMEM((NCH * CH,), jnp.int32),
        pltpu.VMEM((NCH, CH), jnp.int32),
        pltpu.VMEM((NCH, CH), jnp.int32),
        pltpu.VMEM((NCH, CH), jnp.int32),
        pltpu.VMEM((NCH, CH), jnp.float32),
        pltpu.VMEM((NCH, CH), jnp.float32),
        pltpu.VMEM((NCH, CH), jnp.float32),
        pltpu.VMEM((ZCH,), jnp.float32),
        pltpu.VMEM_SHARED((3 * P,), jnp.float32),
        pltpu.SemaphoreType.DMA,
    ],
)(_edge_body)


def _head_body(parts_ref, ba_ref, w1_ref, b1_ref, w2_ref, b2_ref,
               w3_ref, b3_ref, out_ref):
    p = parts_ref[...]
    tot = p[0] + p[1]
    inv = 1.0 / (tot[0:P] + 1e-16)
    o0 = jnp.maximum(tot[P:2 * P] * inv + ba_ref[0], 0.0)
    o1 = jnp.maximum(tot[2 * P:3 * P] * inv + ba_ref[1], 0.0)
    f1 = [jnp.maximum(o0 * w1_ref[0, j] + o1 * w1_ref[1, j] + b1_ref[j], 0.0)
          for j in range(2)]
    f2 = [jnp.maximum(f1[0] * w2_ref[0, j] + f1[1] * w2_ref[1, j] + b2_ref[j],
                      0.0)
          for j in range(4)]
    z = (f2[0] * w3_ref[0, 0] + f2[1] * w3_ref[1, 0]
         + f2[2] * w3_ref[2, 0] + f2[3] * w3_ref[3, 0] + b3_ref[0])
    out_ref[...] = jax.nn.sigmoid(z)[None, :]


def _head(parts, ba, w1, b1, w2, b2, w3, b3):
    smem = pl.BlockSpec(memory_space=pltpu.SMEM)
    return pl.pallas_call(
        _head_body,
        in_specs=[pl.BlockSpec(memory_space=pltpu.VMEM),
                  smem, smem, smem, smem, smem, smem, smem],
        out_specs=pl.BlockSpec(memory_space=pltpu.VMEM),
        out_shape=jax.ShapeDtypeStruct((1, P), jnp.float32),
    )(parts, ba, w1, b1, w2, b2, w3, b3)


def kernel(x_0, x_1, ei_a, ei_b, W_src_a, W_dst_a, att_src_a, att_dst_a,
           bias_a, W_src_b, W_dst_b, att_src_b, att_dst_b, bias_b,
           W_fc1, b_fc1, W_fc2, b_fc2, W_fc3, b_fc3):
    tab_p = _proj(x_0, x_1, W_src_a, W_dst_a, att_src_a, att_dst_a)
    ei = ei_a.astype(jnp.int32)
    parts = _edge_call(tab_p, ei)
    out = _head(parts, bias_a, W_fc1, b_fc1, W_fc2, b_fc2, W_fc3, b_fc3)
    return out[0, :N].reshape(N, 1)
